# baseline (speedup 1.0000x reference)
import functools

import jax
import jax.numpy as jnp
from jax import lax
from jax.experimental import pallas as pl
from jax.experimental.pallas import tpu as pltpu
from jax.experimental.pallas import tpu_sc as plsc

N_NODES = 10000
N_EDGES = 320000
D = 128
ALPHA = 0.5

NS = 16
CHUNK = 128
NCHUNK = 158
EDGES_PAD = NS * NCHUNK * CHUNK
N_PAD = N_NODES + 8
ROWS_PER_TILE = 624
ROWS_TAIL = N_NODES - NS * ROWS_PER_TILE
ACC_TAIL = N_PAD - NS * ROWS_PER_TILE
HIST_W = 16

_mesh = plsc.VectorSubcoreMesh(core_axis_name="c", subcore_axis_name="s")
_sc_params = pltpu.CompilerParams(use_tc_tiling_on_sc=False)


def _for_tile_rows(tile, fn, tail):
    base = tile * ROWS_PER_TILE

    @pl.loop(0, 4)
    def _(c):
        fn(base + c * CHUNK, CHUNK)

    fn(base + 4 * CHUNK, ROWS_PER_TILE - 4 * CHUNK)

    @pl.when(tile == NS - 1)
    def _():
        fn(NS * ROWS_PER_TILE, tail)


def _fill_rows(buf, width, value):
    @pl.loop(0, CHUNK)
    def _(i):
        for j in range(width // 16):
            buf[i, pl.ds(j * 16, 16)] = jnp.full((16,), value, jnp.float32)


@functools.partial(
    pl.kernel,
    out_type=(
        jax.ShapeDtypeStruct((N_NODES, HIST_W), jnp.float32),
        jax.ShapeDtypeStruct((N_NODES, HIST_W), jnp.float32),
    ),
    mesh=_mesh,
    scratch_types=[
        pltpu.VMEM((NCHUNK, CHUNK), jnp.int32),
        pltpu.VMEM((CHUNK, HIST_W), jnp.float32),
        pltpu.VMEM((CHUNK, HIST_W), jnp.float32),
        pltpu.VMEM_SHARED((N_PAD, HIST_W), jnp.float32),
    ],
    compiler_params=_sc_params,
)
def _degree_sc(row_hbm, col_hbm, degf_hbm, degb_hbm, idx_v, ones_v, zero_v,
               hist):
    core = lax.axis_index("c")
    tile = lax.axis_index("s")

    _fill_rows(ones_v, HIST_W, 1.0)
    _fill_rows(zero_v, HIST_W, 0.0)
    _for_tile_rows(
        tile,
        lambda off, sz: pltpu.sync_copy(zero_v.at[pl.ds(0, sz)],
                                        hist.at[pl.ds(off, sz)]),
        ACC_TAIL)

    @pl.when(core == 0)
    def _():
        pltpu.sync_copy(row_hbm.at[tile], idx_v)

    @pl.when(core == 1)
    def _():
        pltpu.sync_copy(col_hbm.at[tile], idx_v)

    plsc.subcore_barrier()

    @pl.loop(0, NCHUNK)
    def _(c):
        pltpu.sync_copy(ones_v, hist.at[idx_v.at[c]], add=True)

    plsc.subcore_barrier()

    def _writeout(out_hbm):
        def fn(off, sz):
            pltpu.sync_copy(hist.at[pl.ds(off, sz)], zero_v.at[pl.ds(0, sz)])
            pltpu.sync_copy(zero_v.at[pl.ds(0, sz)], out_hbm.at[pl.ds(off, sz)])

        _for_tile_rows(tile, fn, ROWS_TAIL)

    @pl.when(core == 0)
    def _():
        _writeout(degf_hbm)

    @pl.when(core == 1)
    def _():
        _writeout(degb_hbm)


DH = D // 2


@functools.partial(
    pl.kernel,
    out_type=(
        jax.ShapeDtypeStruct((N_NODES, DH), jnp.float32),
        jax.ShapeDtypeStruct((N_NODES, DH), jnp.float32),
    ),
    mesh=_mesh,
    scratch_types=[
        pltpu.VMEM((NCHUNK, CHUNK), jnp.int32),
        pltpu.VMEM((NCHUNK, CHUNK), jnp.int32),
        pltpu.VMEM((CHUNK, DH), jnp.float32),
        pltpu.VMEM((CHUNK, DH), jnp.float32),
        pltpu.VMEM_SHARED((N_PAD, DH), jnp.float32),
        pltpu.SemaphoreType.DMA,
        pltpu.SemaphoreType.DMA,
    ],
    compiler_params=_sc_params,
)
def _spmm_sc(row_hbm, col_hbm, xs_hbm, xd_hbm, outf_hbm, outb_hbm,
             row_v, col_v, buf_a, buf_b, acc, sem_a, sem_b):
    core = lax.axis_index("c")
    tile = lax.axis_index("s")

    _fill_rows(buf_a, DH, 0.0)
    _for_tile_rows(
        tile,
        lambda off, sz: pltpu.sync_copy(buf_a.at[pl.ds(0, sz)],
                                        acc.at[pl.ds(off, sz)]),
        ACC_TAIL)
    pltpu.sync_copy(row_hbm.at[tile], row_v)
    pltpu.sync_copy(col_hbm.at[tile], col_v)
    plsc.subcore_barrier()

    def run_direction(src_hbm, g_idx, s_idx):
        def issue(c, buf, sem):
            pltpu.make_async_copy(src_hbm.at[g_idx.at[c]], buf, sem).start()

        def wait(c, buf, sem):
            pltpu.make_async_copy(src_hbm.at[g_idx.at[c]], buf, sem).wait()

        issue(0, buf_a, sem_a)

        @pl.loop(0, NCHUNK, step=2)
        def _(c):
            issue(c + 1, buf_b, sem_b)
            wait(c, buf_a, sem_a)
            pltpu.sync_copy(buf_a, acc.at[s_idx.at[c]], add=True)

            @pl.when(c + 2 < NCHUNK)
            def _():
                issue(c + 2, buf_a, sem_a)

            wait(c + 1, buf_b, sem_b)
            pltpu.sync_copy(buf_b, acc.at[s_idx.at[c + 1]], add=True)

    @pl.when(core == 0)
    def _():
        run_direction(xs_hbm, col_v, row_v)

    @pl.when(core == 1)
    def _():
        run_direction(xd_hbm, row_v, col_v)

    plsc.subcore_barrier()

    def _writeout(out_hbm):
        def fn(off, sz):
            pltpu.sync_copy(acc.at[pl.ds(off, sz)], buf_a.at[pl.ds(0, sz)])
            pltpu.sync_copy(buf_a.at[pl.ds(0, sz)], out_hbm.at[pl.ds(off, sz)])

        _for_tile_rows(tile, fn, ROWS_TAIL)

    @pl.when(core == 0)
    def _():
        _writeout(outf_hbm)

    @pl.when(core == 1)
    def _():
        _writeout(outb_hbm)


def _matmul_tc(x, w_sd, w_ds):
    def body(x_ref, wsd_ref, wds_ref, xs_ref, xd_ref):
        xb = x_ref[...]
        dn = (((1,), (1,)), ((), ()))
        xs_ref[...] = ALPHA * lax.dot_general(
            xb, wsd_ref[...], dn, preferred_element_type=jnp.float32)
        xd_ref[...] = (1.0 - ALPHA) * lax.dot_general(
            xb, wds_ref[...], dn, preferred_element_type=jnp.float32)

    blk = N_NODES // 10
    return pl.pallas_call(
        body,
        grid=(10,),
        in_specs=[
            pl.BlockSpec((blk, D), lambda i: (i, 0)),
            pl.BlockSpec((D, D), lambda i: (0, 0)),
            pl.BlockSpec((D, D), lambda i: (0, 0)),
        ],
        out_specs=[
            pl.BlockSpec((blk, D), lambda i: (i, 0)),
            pl.BlockSpec((blk, D), lambda i: (i, 0)),
        ],
        out_shape=[jax.ShapeDtypeStruct((N_NODES, D), jnp.float32)] * 2,
    )(x, w_sd, w_ds)


def _dinv(deg_block):
    return jnp.where(deg_block > 0,
                     lax.rsqrt(jnp.maximum(deg_block, 1e-12)),
                     0.0)


def _scale_tc(xs0, xd0, degf, degb):
    def body(xs_ref, xd_ref, df_ref, db_ref, xsl_ref, xsh_ref, xdl_ref,
             xdh_ref):
        xs = _dinv(df_ref[:, 0:1]) * xs_ref[...]
        xd = _dinv(db_ref[:, 0:1]) * xd_ref[...]
        xsl_ref[...] = xs[:, :DH]
        xsh_ref[...] = xs[:, DH:]
        xdl_ref[...] = xd[:, :DH]
        xdh_ref[...] = xd[:, DH:]

    blk = N_NODES // 10
    return pl.pallas_call(
        body,
        grid=(10,),
        in_specs=[
            pl.BlockSpec((blk, D), lambda i: (i, 0)),
            pl.BlockSpec((blk, D), lambda i: (i, 0)),
            pl.BlockSpec((blk, HIST_W), lambda i: (i, 0)),
            pl.BlockSpec((blk, HIST_W), lambda i: (i, 0)),
        ],
        out_specs=[pl.BlockSpec((blk, DH), lambda i: (i, 0))] * 4,
        out_shape=[jax.ShapeDtypeStruct((N_PAD, DH), jnp.float32)] * 4,
    )(xs0, xd0, degf, degb)


def _combine_tc(afl, afh, abl, abh, degf, degb, bsd, bds):
    def body(afl_ref, afh_ref, abl_ref, abh_ref, df_ref, db_ref, bsd_ref,
             bds_ref, o_ref):
        bias = ALPHA * bsd_ref[0:1, :] + (1.0 - ALPHA) * bds_ref[0:1, :]
        dif = _dinv(df_ref[:, 0:1])
        dib = _dinv(db_ref[:, 0:1])
        af = jnp.concatenate([afl_ref[...], afh_ref[...]], axis=1)
        ab = jnp.concatenate([abl_ref[...], abh_ref[...]], axis=1)
        o_ref[...] = dif * af + dib * ab + bias

    blk = N_NODES // 10
    return pl.pallas_call(
        body,
        grid=(10,),
        in_specs=[
            pl.BlockSpec((blk, DH), lambda i: (i, 0)),
            pl.BlockSpec((blk, DH), lambda i: (i, 0)),
            pl.BlockSpec((blk, DH), lambda i: (i, 0)),
            pl.BlockSpec((blk, DH), lambda i: (i, 0)),
            pl.BlockSpec((blk, HIST_W), lambda i: (i, 0)),
            pl.BlockSpec((blk, HIST_W), lambda i: (i, 0)),
            pl.BlockSpec((8, D), lambda i: (0, 0)),
            pl.BlockSpec((8, D), lambda i: (0, 0)),
        ],
        out_specs=pl.BlockSpec((blk, D), lambda i: (i, 0)),
        out_shape=jax.ShapeDtypeStruct((N_NODES, D), jnp.float32),
    )(afl, afh, abl, abh, degf, degb, bsd, bds)


@jax.jit
def kernel(x, edge_index, W_sd, b_sd, W_ds, b_ds):
    pad = jnp.full((EDGES_PAD - N_EDGES,), N_NODES, jnp.int32)
    row = jnp.concatenate([edge_index[0], pad]).reshape(NS, NCHUNK, CHUNK)
    col = jnp.concatenate([edge_index[1], pad]).reshape(NS, NCHUNK, CHUNK)

    degf, degb = _degree_sc(row, col)
    xs0, xd0 = _matmul_tc(x, W_sd, W_ds)
    xsl, xsh, xdl, xdh = _scale_tc(xs0, xd0, degf, degb)
    afl, abl = _spmm_sc(row, col, xsl, xdl)
    afh, abh = _spmm_sc(row, col, xsh, xdh)

    bsd = jnp.broadcast_to(b_sd[None, :], (8, D))
    bds = jnp.broadcast_to(b_ds[None, :], (8, D))
    return _combine_tc(afl, afh, abl, abh, degf, degb, bsd, bds)


# speedup vs baseline: 1.0168x; 1.0168x over previous
import functools

import jax
import jax.numpy as jnp
from jax import lax
from jax.experimental import pallas as pl
from jax.experimental.pallas import tpu as pltpu
from jax.experimental.pallas import tpu_sc as plsc

N_NODES = 10000
N_EDGES = 320000
D = 128
ALPHA = 0.5

NS = 16
CHUNK = 64
NCHUNK = 314
EDGES_PAD = NS * NCHUNK * CHUNK
N_PAD = N_NODES + 8
ROWS_PER_TILE = 624
ROWS_TAIL = N_NODES - NS * ROWS_PER_TILE
ACC_TAIL = N_PAD - NS * ROWS_PER_TILE
HIST_W = 16

_mesh = plsc.VectorSubcoreMesh(core_axis_name="c", subcore_axis_name="s")
_sc_params = pltpu.CompilerParams(use_tc_tiling_on_sc=False)


def _for_tile_rows(tile, fn, tail):
    base = tile * ROWS_PER_TILE
    n_full = ROWS_PER_TILE // CHUNK
    rem = ROWS_PER_TILE % CHUNK

    @pl.loop(0, n_full)
    def _(c):
        fn(base + c * CHUNK, CHUNK)

    if rem:
        fn(base + n_full * CHUNK, rem)

    @pl.when(tile == NS - 1)
    def _():
        fn(NS * ROWS_PER_TILE, tail)


def _fill_rows(buf, width, value):
    @pl.loop(0, CHUNK)
    def _(i):
        for j in range(width // 16):
            buf[i, pl.ds(j * 16, 16)] = jnp.full((16,), value, jnp.float32)


@functools.partial(
    pl.kernel,
    out_type=(
        jax.ShapeDtypeStruct((N_NODES, HIST_W), jnp.float32),
        jax.ShapeDtypeStruct((N_NODES, HIST_W), jnp.float32),
    ),
    mesh=_mesh,
    scratch_types=[
        pltpu.VMEM((NCHUNK, CHUNK), jnp.int32),
        pltpu.VMEM((CHUNK, HIST_W), jnp.float32),
        pltpu.VMEM((CHUNK, HIST_W), jnp.float32),
        pltpu.VMEM_SHARED((N_PAD, HIST_W), jnp.float32),
    ],
    compiler_params=_sc_params,
)
def _degree_sc(row_hbm, col_hbm, degf_hbm, degb_hbm, idx_v, ones_v, zero_v,
               hist):
    core = lax.axis_index("c")
    tile = lax.axis_index("s")

    _fill_rows(ones_v, HIST_W, 1.0)
    _fill_rows(zero_v, HIST_W, 0.0)
    _for_tile_rows(
        tile,
        lambda off, sz: pltpu.sync_copy(zero_v.at[pl.ds(0, sz)],
                                        hist.at[pl.ds(off, sz)]),
        ACC_TAIL)

    @pl.when(core == 0)
    def _():
        pltpu.sync_copy(row_hbm.at[tile], idx_v)

    @pl.when(core == 1)
    def _():
        pltpu.sync_copy(col_hbm.at[tile], idx_v)

    plsc.subcore_barrier()

    @pl.loop(0, NCHUNK)
    def _(c):
        pltpu.sync_copy(ones_v, hist.at[idx_v.at[c]], add=True)

    plsc.subcore_barrier()

    def _writeout(out_hbm):
        def fn(off, sz):
            pltpu.sync_copy(hist.at[pl.ds(off, sz)], zero_v.at[pl.ds(0, sz)])
            pltpu.sync_copy(zero_v.at[pl.ds(0, sz)], out_hbm.at[pl.ds(off, sz)])

        _for_tile_rows(tile, fn, ROWS_TAIL)

    @pl.when(core == 0)
    def _():
        _writeout(degf_hbm)

    @pl.when(core == 1)
    def _():
        _writeout(degb_hbm)


DH = D // 2


@functools.partial(
    pl.kernel,
    out_type=(
        jax.ShapeDtypeStruct((N_NODES, DH), jnp.float32),
        jax.ShapeDtypeStruct((N_NODES, DH), jnp.float32),
    ),
    mesh=_mesh,
    scratch_types=[
        pltpu.VMEM((NCHUNK, CHUNK), jnp.int32),
        pltpu.VMEM((NCHUNK, CHUNK), jnp.int32),
        pltpu.VMEM((CHUNK, DH), jnp.float32),
        pltpu.VMEM((CHUNK, DH), jnp.float32),
        pltpu.VMEM_SHARED((N_PAD, DH), jnp.float32),
        pltpu.SemaphoreType.DMA,
        pltpu.SemaphoreType.DMA,
    ],
    compiler_params=_sc_params,
)
def _spmm_sc(row_hbm, col_hbm, xs_hbm, xd_hbm, outf_hbm, outb_hbm,
             row_v, col_v, buf_a, buf_b, acc, sem_a, sem_b):
    core = lax.axis_index("c")
    tile = lax.axis_index("s")

    _fill_rows(buf_a, DH, 0.0)
    _for_tile_rows(
        tile,
        lambda off, sz: pltpu.sync_copy(buf_a.at[pl.ds(0, sz)],
                                        acc.at[pl.ds(off, sz)]),
        ACC_TAIL)
    pltpu.sync_copy(row_hbm.at[tile], row_v)
    pltpu.sync_copy(col_hbm.at[tile], col_v)
    plsc.subcore_barrier()

    def run_direction(src_hbm, g_idx, s_idx):
        def issue(c, buf, sem):
            pltpu.make_async_copy(src_hbm.at[g_idx.at[c]], buf, sem).start()

        def wait(c, buf, sem):
            pltpu.make_async_copy(src_hbm.at[g_idx.at[c]], buf, sem).wait()

        issue(0, buf_a, sem_a)

        @pl.loop(0, NCHUNK, step=2)
        def _(c):
            issue(c + 1, buf_b, sem_b)
            wait(c, buf_a, sem_a)
            pltpu.sync_copy(buf_a, acc.at[s_idx.at[c]], add=True)

            @pl.when(c + 2 < NCHUNK)
            def _():
                issue(c + 2, buf_a, sem_a)

            wait(c + 1, buf_b, sem_b)
            pltpu.sync_copy(buf_b, acc.at[s_idx.at[c + 1]], add=True)

    @pl.when(core == 0)
    def _():
        run_direction(xs_hbm, col_v, row_v)

    @pl.when(core == 1)
    def _():
        run_direction(xd_hbm, row_v, col_v)

    plsc.subcore_barrier()

    def _writeout(out_hbm):
        def fn(off, sz):
            pltpu.sync_copy(acc.at[pl.ds(off, sz)], buf_a.at[pl.ds(0, sz)])
            pltpu.sync_copy(buf_a.at[pl.ds(0, sz)], out_hbm.at[pl.ds(off, sz)])

        _for_tile_rows(tile, fn, ROWS_TAIL)

    @pl.when(core == 0)
    def _():
        _writeout(outf_hbm)

    @pl.when(core == 1)
    def _():
        _writeout(outb_hbm)


def _matmul_tc(x, w_sd, w_ds):
    def body(x_ref, wsd_ref, wds_ref, xs_ref, xd_ref):
        xb = x_ref[...]
        dn = (((1,), (1,)), ((), ()))
        xs_ref[...] = ALPHA * lax.dot_general(
            xb, wsd_ref[...], dn, preferred_element_type=jnp.float32)
        xd_ref[...] = (1.0 - ALPHA) * lax.dot_general(
            xb, wds_ref[...], dn, preferred_element_type=jnp.float32)

    blk = N_NODES // 10
    return pl.pallas_call(
        body,
        grid=(10,),
        in_specs=[
            pl.BlockSpec((blk, D), lambda i: (i, 0)),
            pl.BlockSpec((D, D), lambda i: (0, 0)),
            pl.BlockSpec((D, D), lambda i: (0, 0)),
        ],
        out_specs=[
            pl.BlockSpec((blk, D), lambda i: (i, 0)),
            pl.BlockSpec((blk, D), lambda i: (i, 0)),
        ],
        out_shape=[jax.ShapeDtypeStruct((N_NODES, D), jnp.float32)] * 2,
    )(x, w_sd, w_ds)


def _dinv(deg_block):
    return jnp.where(deg_block > 0,
                     lax.rsqrt(jnp.maximum(deg_block, 1e-12)),
                     0.0)


def _scale_tc(xs0, xd0, degf, degb):
    def body(xs_ref, xd_ref, df_ref, db_ref, xsl_ref, xsh_ref, xdl_ref,
             xdh_ref):
        xs = _dinv(df_ref[:, 0:1]) * xs_ref[...]
        xd = _dinv(db_ref[:, 0:1]) * xd_ref[...]
        xsl_ref[...] = xs[:, :DH]
        xsh_ref[...] = xs[:, DH:]
        xdl_ref[...] = xd[:, :DH]
        xdh_ref[...] = xd[:, DH:]

    blk = N_NODES // 10
    return pl.pallas_call(
        body,
        grid=(10,),
        in_specs=[
            pl.BlockSpec((blk, D), lambda i: (i, 0)),
            pl.BlockSpec((blk, D), lambda i: (i, 0)),
            pl.BlockSpec((blk, HIST_W), lambda i: (i, 0)),
            pl.BlockSpec((blk, HIST_W), lambda i: (i, 0)),
        ],
        out_specs=[pl.BlockSpec((blk, DH), lambda i: (i, 0))] * 4,
        out_shape=[jax.ShapeDtypeStruct((N_PAD, DH), jnp.float32)] * 4,
    )(xs0, xd0, degf, degb)


def _combine_tc(afl, afh, abl, abh, degf, degb, bsd, bds):
    def body(afl_ref, afh_ref, abl_ref, abh_ref, df_ref, db_ref, bsd_ref,
             bds_ref, o_ref):
        bias = ALPHA * bsd_ref[0:1, :] + (1.0 - ALPHA) * bds_ref[0:1, :]
        dif = _dinv(df_ref[:, 0:1])
        dib = _dinv(db_ref[:, 0:1])
        af = jnp.concatenate([afl_ref[...], afh_ref[...]], axis=1)
        ab = jnp.concatenate([abl_ref[...], abh_ref[...]], axis=1)
        o_ref[...] = dif * af + dib * ab + bias

    blk = N_NODES // 10
    return pl.pallas_call(
        body,
        grid=(10,),
        in_specs=[
            pl.BlockSpec((blk, DH), lambda i: (i, 0)),
            pl.BlockSpec((blk, DH), lambda i: (i, 0)),
            pl.BlockSpec((blk, DH), lambda i: (i, 0)),
            pl.BlockSpec((blk, DH), lambda i: (i, 0)),
            pl.BlockSpec((blk, HIST_W), lambda i: (i, 0)),
            pl.BlockSpec((blk, HIST_W), lambda i: (i, 0)),
            pl.BlockSpec((8, D), lambda i: (0, 0)),
            pl.BlockSpec((8, D), lambda i: (0, 0)),
        ],
        out_specs=pl.BlockSpec((blk, D), lambda i: (i, 0)),
        out_shape=jax.ShapeDtypeStruct((N_NODES, D), jnp.float32),
    )(afl, afh, abl, abh, degf, degb, bsd, bds)


@jax.jit
def kernel(x, edge_index, W_sd, b_sd, W_ds, b_ds):
    pad = jnp.full((EDGES_PAD - N_EDGES,), N_NODES, jnp.int32)
    row = jnp.concatenate([edge_index[0], pad]).reshape(NS, NCHUNK, CHUNK)
    col = jnp.concatenate([edge_index[1], pad]).reshape(NS, NCHUNK, CHUNK)

    degf, degb = _degree_sc(row, col)
    xs0, xd0 = _matmul_tc(x, W_sd, W_ds)
    xsl, xsh, xdl, xdh = _scale_tc(xs0, xd0, degf, degb)
    afl, abl = _spmm_sc(row, col, xsl, xdl)
    afh, abh = _spmm_sc(row, col, xsh, xdh)

    bsd = jnp.broadcast_to(b_sd[None, :], (8, D))
    bds = jnp.broadcast_to(b_ds[None, :], (8, D))
    return _combine_tc(afl, afh, abl, abh, degf, degb, bsd, bds)


# speedup vs baseline: 1.1066x; 1.0883x over previous
import functools

import jax
import jax.numpy as jnp
from jax import lax
from jax.experimental import pallas as pl
from jax.experimental.pallas import tpu as pltpu
from jax.experimental.pallas import tpu_sc as plsc

N_NODES = 10000
N_EDGES = 320000
D = 128
ALPHA = 0.5

NS = 16
CHUNK = 80
EDGES_PER_TILE = N_EDGES // NS
NCHUNK = EDGES_PER_TILE // CHUNK
ROWS_PER_TILE = 624
ROWS_TAIL = N_NODES - NS * ROWS_PER_TILE
HIST_W = 16

_mesh = plsc.VectorSubcoreMesh(core_axis_name="c", subcore_axis_name="s")
_sc_params = pltpu.CompilerParams(use_tc_tiling_on_sc=False)


def _for_tile_rows(tile, fn):
    base = tile * ROWS_PER_TILE

    @pl.loop(0, 7)
    def _(c):
        fn(base + c * CHUNK, CHUNK)

    fn(base + 7 * CHUNK, ROWS_PER_TILE - 7 * CHUNK)

    @pl.when(tile == NS - 1)
    def _():
        fn(NS * ROWS_PER_TILE, ROWS_TAIL)


def _fill_rows(buf, width, value):
    @pl.loop(0, CHUNK)
    def _(i):
        for j in range(width // 16):
            buf[i, pl.ds(j * 16, 16)] = jnp.full((16,), value, jnp.float32)


@functools.partial(
    pl.kernel,
    out_type=(
        jax.ShapeDtypeStruct((N_NODES, HIST_W), jnp.float32),
        jax.ShapeDtypeStruct((N_NODES, HIST_W), jnp.float32),
    ),
    mesh=_mesh,
    scratch_types=[
        pltpu.VMEM((NCHUNK, CHUNK), jnp.int32),
        pltpu.VMEM((CHUNK, HIST_W), jnp.float32),
        pltpu.VMEM((CHUNK, HIST_W), jnp.float32),
        pltpu.VMEM_SHARED((N_NODES, HIST_W), jnp.float32),
    ],
    compiler_params=_sc_params,
)
def _degree_sc(row_hbm, col_hbm, degf_hbm, degb_hbm, idx_v, ones_v, zero_v,
               hist):
    core = lax.axis_index("c")
    tile = lax.axis_index("s")

    _fill_rows(ones_v, HIST_W, 1.0)
    _fill_rows(zero_v, HIST_W, 0.0)
    _for_tile_rows(
        tile,
        lambda off, sz: pltpu.sync_copy(zero_v.at[pl.ds(0, sz)],
                                        hist.at[pl.ds(off, sz)]))

    @pl.when(core == 0)
    def _():
        pltpu.sync_copy(row_hbm.at[tile], idx_v)

    @pl.when(core == 1)
    def _():
        pltpu.sync_copy(col_hbm.at[tile], idx_v)

    plsc.subcore_barrier()

    @pl.loop(0, NCHUNK)
    def _(c):
        pltpu.sync_copy(ones_v, hist.at[idx_v.at[c]], add=True)

    plsc.subcore_barrier()

    def _writeout(out_hbm):
        def fn(off, sz):
            pltpu.sync_copy(hist.at[pl.ds(off, sz)], zero_v.at[pl.ds(0, sz)])
            pltpu.sync_copy(zero_v.at[pl.ds(0, sz)], out_hbm.at[pl.ds(off, sz)])

        _for_tile_rows(tile, fn)

    @pl.when(core == 0)
    def _():
        _writeout(degf_hbm)

    @pl.when(core == 1)
    def _():
        _writeout(degb_hbm)


DH = D // 2


@functools.partial(
    pl.kernel,
    out_type=(
        jax.ShapeDtypeStruct((N_NODES, DH), jnp.float32),
        jax.ShapeDtypeStruct((N_NODES, DH), jnp.float32),
    ),
    mesh=_mesh,
    scratch_types=[
        pltpu.VMEM((NCHUNK, CHUNK), jnp.int32),
        pltpu.VMEM((NCHUNK, CHUNK), jnp.int32),
        pltpu.VMEM((CHUNK, DH), jnp.float32),
        pltpu.VMEM((CHUNK, DH), jnp.float32),
        pltpu.VMEM_SHARED((N_NODES, DH), jnp.float32),
        pltpu.SemaphoreType.DMA,
        pltpu.SemaphoreType.DMA,
        pltpu.SemaphoreType.DMA,
        pltpu.SemaphoreType.DMA,
    ],
    compiler_params=_sc_params,
)
def _spmm_sc(row_hbm, col_hbm, xs_hbm, xd_hbm, outf_hbm, outb_hbm,
             row_v, col_v, buf_a, buf_b, acc, sem_ga, sem_gb, sem_sa, sem_sb):
    core = lax.axis_index("c")
    tile = lax.axis_index("s")

    _fill_rows(buf_a, DH, 0.0)
    _for_tile_rows(
        tile,
        lambda off, sz: pltpu.sync_copy(buf_a.at[pl.ds(0, sz)],
                                        acc.at[pl.ds(off, sz)]))
    pltpu.sync_copy(row_hbm.at[tile], row_v)
    pltpu.sync_copy(col_hbm.at[tile], col_v)
    plsc.subcore_barrier()

    def run_direction(src_hbm, g_idx, s_idx):
        def g_issue(c, buf, sem):
            pltpu.make_async_copy(src_hbm.at[g_idx.at[c]], buf, sem).start()

        def g_wait(c, buf, sem):
            pltpu.make_async_copy(src_hbm.at[g_idx.at[c]], buf, sem).wait()

        def s_issue(c, buf, sem):
            pltpu.async_copy(buf, acc.at[s_idx.at[c]], sem, add=True)

        def s_wait(c, buf, sem):
            pltpu.make_async_copy(buf, acc.at[s_idx.at[c]], sem).wait()

        g_issue(0, buf_a, sem_ga)
        g_issue(1, buf_b, sem_gb)

        @pl.loop(0, NCHUNK, step=2)
        def _(c):
            g_wait(c, buf_a, sem_ga)
            s_issue(c, buf_a, sem_sa)
            g_wait(c + 1, buf_b, sem_gb)
            s_issue(c + 1, buf_b, sem_sb)

            @pl.when(c + 2 < NCHUNK)
            def _():
                s_wait(c, buf_a, sem_sa)
                g_issue(c + 2, buf_a, sem_ga)

            @pl.when(c + 3 < NCHUNK)
            def _():
                s_wait(c + 1, buf_b, sem_sb)
                g_issue(c + 3, buf_b, sem_gb)

        s_wait(NCHUNK - 2, buf_a, sem_sa)
        s_wait(NCHUNK - 1, buf_b, sem_sb)

    @pl.when(core == 0)
    def _():
        run_direction(xs_hbm, col_v, row_v)

    @pl.when(core == 1)
    def _():
        run_direction(xd_hbm, row_v, col_v)

    plsc.subcore_barrier()

    def _writeout(out_hbm):
        def fn(off, sz):
            pltpu.sync_copy(acc.at[pl.ds(off, sz)], buf_a.at[pl.ds(0, sz)])
            pltpu.sync_copy(buf_a.at[pl.ds(0, sz)], out_hbm.at[pl.ds(off, sz)])

        _for_tile_rows(tile, fn)

    @pl.when(core == 0)
    def _():
        _writeout(outf_hbm)

    @pl.when(core == 1)
    def _():
        _writeout(outb_hbm)


def _matmul_tc(x, w_sd, w_ds):
    def body(x_ref, wsd_ref, wds_ref, xs_ref, xd_ref):
        xb = x_ref[...]
        dn = (((1,), (1,)), ((), ()))
        xs_ref[...] = ALPHA * lax.dot_general(
            xb, wsd_ref[...], dn, preferred_element_type=jnp.float32)
        xd_ref[...] = (1.0 - ALPHA) * lax.dot_general(
            xb, wds_ref[...], dn, preferred_element_type=jnp.float32)

    blk = N_NODES // 10
    return pl.pallas_call(
        body,
        grid=(10,),
        in_specs=[
            pl.BlockSpec((blk, D), lambda i: (i, 0)),
            pl.BlockSpec((D, D), lambda i: (0, 0)),
            pl.BlockSpec((D, D), lambda i: (0, 0)),
        ],
        out_specs=[
            pl.BlockSpec((blk, D), lambda i: (i, 0)),
            pl.BlockSpec((blk, D), lambda i: (i, 0)),
        ],
        out_shape=[jax.ShapeDtypeStruct((N_NODES, D), jnp.float32)] * 2,
    )(x, w_sd, w_ds)


def _dinv(deg_block):
    return jnp.where(deg_block > 0,
                     lax.rsqrt(jnp.maximum(deg_block, 1e-12)),
                     0.0)


def _scale_tc(xs0, xd0, degf, degb):
    def body(xs_ref, xd_ref, df_ref, db_ref, xsl_ref, xsh_ref, xdl_ref,
             xdh_ref):
        xs = _dinv(df_ref[:, 0:1]) * xs_ref[...]
        xd = _dinv(db_ref[:, 0:1]) * xd_ref[...]
        xsl_ref[...] = xs[:, :DH]
        xsh_ref[...] = xs[:, DH:]
        xdl_ref[...] = xd[:, :DH]
        xdh_ref[...] = xd[:, DH:]

    blk = N_NODES // 10
    return pl.pallas_call(
        body,
        grid=(10,),
        in_specs=[
            pl.BlockSpec((blk, D), lambda i: (i, 0)),
            pl.BlockSpec((blk, D), lambda i: (i, 0)),
            pl.BlockSpec((blk, HIST_W), lambda i: (i, 0)),
            pl.BlockSpec((blk, HIST_W), lambda i: (i, 0)),
        ],
        out_specs=[pl.BlockSpec((blk, DH), lambda i: (i, 0))] * 4,
        out_shape=[jax.ShapeDtypeStruct((N_NODES, DH), jnp.float32)] * 4,
    )(xs0, xd0, degf, degb)


def _combine_tc(afl, afh, abl, abh, degf, degb, bsd, bds):
    def body(afl_ref, afh_ref, abl_ref, abh_ref, df_ref, db_ref, bsd_ref,
             bds_ref, o_ref):
        bias = ALPHA * bsd_ref[0:1, :] + (1.0 - ALPHA) * bds_ref[0:1, :]
        dif = _dinv(df_ref[:, 0:1])
        dib = _dinv(db_ref[:, 0:1])
        af = jnp.concatenate([afl_ref[...], afh_ref[...]], axis=1)
        ab = jnp.concatenate([abl_ref[...], abh_ref[...]], axis=1)
        o_ref[...] = dif * af + dib * ab + bias

    blk = N_NODES // 10
    return pl.pallas_call(
        body,
        grid=(10,),
        in_specs=[
            pl.BlockSpec((blk, DH), lambda i: (i, 0)),
            pl.BlockSpec((blk, DH), lambda i: (i, 0)),
            pl.BlockSpec((blk, DH), lambda i: (i, 0)),
            pl.BlockSpec((blk, DH), lambda i: (i, 0)),
            pl.BlockSpec((blk, HIST_W), lambda i: (i, 0)),
            pl.BlockSpec((blk, HIST_W), lambda i: (i, 0)),
            pl.BlockSpec((8, D), lambda i: (0, 0)),
            pl.BlockSpec((8, D), lambda i: (0, 0)),
        ],
        out_specs=pl.BlockSpec((blk, D), lambda i: (i, 0)),
        out_shape=jax.ShapeDtypeStruct((N_NODES, D), jnp.float32),
    )(afl, afh, abl, abh, degf, degb, bsd, bds)


@jax.jit
def kernel(x, edge_index, W_sd, b_sd, W_ds, b_ds):
    row = edge_index[0].reshape(NS, NCHUNK, CHUNK)
    col = edge_index[1].reshape(NS, NCHUNK, CHUNK)

    degf, degb = _degree_sc(row, col)
    xs0, xd0 = _matmul_tc(x, W_sd, W_ds)
    xsl, xsh, xdl, xdh = _scale_tc(xs0, xd0, degf, degb)
    afl, abl = _spmm_sc(row, col, xsl, xdl)
    afh, abh = _spmm_sc(row, col, xsh, xdh)

    bsd = jnp.broadcast_to(b_sd[None, :], (8, D))
    bds = jnp.broadcast_to(b_ds[None, :], (8, D))
    return _combine_tc(afl, afh, abl, abh, degf, degb, bsd, bds)


# speedup vs baseline: 1.2438x; 1.1240x over previous
import functools

import jax
import jax.numpy as jnp
from jax import lax
from jax.experimental import pallas as pl
from jax.experimental.pallas import tpu as pltpu
from jax.experimental.pallas import tpu_sc as plsc

N_NODES = 10000
N_EDGES = 320000
D = 128
ALPHA = 0.5

NS = 16
CHUNK = 80
EDGES_PER_TILE = N_EDGES // NS
NCHUNK = EDGES_PER_TILE // CHUNK
ROWS_PER_TILE = 624
ROWS_TAIL = N_NODES - NS * ROWS_PER_TILE
HIST_W = 16

_mesh = plsc.VectorSubcoreMesh(core_axis_name="c", subcore_axis_name="s")
_sc_params = pltpu.CompilerParams(use_tc_tiling_on_sc=False)


def _for_tile_rows(tile, fn):
    base = tile * ROWS_PER_TILE

    @pl.loop(0, 7)
    def _(c):
        fn(base + c * CHUNK, CHUNK)

    fn(base + 7 * CHUNK, ROWS_PER_TILE - 7 * CHUNK)

    @pl.when(tile == NS - 1)
    def _():
        fn(NS * ROWS_PER_TILE, ROWS_TAIL)


def _fill_rows(buf, width, value):
    @pl.loop(0, CHUNK)
    def _(i):
        for j in range(width // 16):
            buf[i, pl.ds(j * 16, 16)] = jnp.full((16,), value, jnp.float32)


@functools.partial(
    pl.kernel,
    out_type=(
        jax.ShapeDtypeStruct((N_NODES, HIST_W), jnp.float32),
        jax.ShapeDtypeStruct((N_NODES, HIST_W), jnp.float32),
    ),
    mesh=_mesh,
    scratch_types=[
        pltpu.VMEM((NCHUNK, CHUNK), jnp.int32),
        pltpu.VMEM((CHUNK, HIST_W), jnp.float32),
        pltpu.VMEM((CHUNK, HIST_W), jnp.float32),
        pltpu.VMEM_SHARED((N_NODES, HIST_W), jnp.float32),
    ],
    compiler_params=_sc_params,
)
def _degree_sc(row_hbm, col_hbm, degf_hbm, degb_hbm, idx_v, ones_v, zero_v,
               hist):
    core = lax.axis_index("c")
    tile = lax.axis_index("s")

    _fill_rows(ones_v, HIST_W, 1.0)
    _fill_rows(zero_v, HIST_W, 0.0)
    _for_tile_rows(
        tile,
        lambda off, sz: pltpu.sync_copy(zero_v.at[pl.ds(0, sz)],
                                        hist.at[pl.ds(off, sz)]))

    @pl.when(core == 0)
    def _():
        pltpu.sync_copy(row_hbm.at[tile], idx_v)

    @pl.when(core == 1)
    def _():
        pltpu.sync_copy(col_hbm.at[tile], idx_v)

    plsc.subcore_barrier()

    @pl.loop(0, NCHUNK)
    def _(c):
        pltpu.sync_copy(ones_v, hist.at[idx_v.at[c]], add=True)

    plsc.subcore_barrier()

    def _writeout(out_hbm):
        def fn(off, sz):
            pltpu.sync_copy(hist.at[pl.ds(off, sz)], zero_v.at[pl.ds(0, sz)])
            pltpu.sync_copy(zero_v.at[pl.ds(0, sz)], out_hbm.at[pl.ds(off, sz)])

        _for_tile_rows(tile, fn)

    @pl.when(core == 0)
    def _():
        _writeout(degf_hbm)

    @pl.when(core == 1)
    def _():
        _writeout(degb_hbm)


DH = D // 2


@functools.partial(
    pl.kernel,
    out_type=(
        jax.ShapeDtypeStruct((N_NODES, DH), jnp.float32),
        jax.ShapeDtypeStruct((N_NODES, DH), jnp.float32),
        jax.ShapeDtypeStruct((N_NODES, DH), jnp.float32),
        jax.ShapeDtypeStruct((N_NODES, DH), jnp.float32),
    ),
    mesh=_mesh,
    scratch_types=[
        pltpu.VMEM((NCHUNK, CHUNK), jnp.int32),
        pltpu.VMEM((NCHUNK, CHUNK), jnp.int32),
        pltpu.VMEM((CHUNK, DH), jnp.float32),
        pltpu.VMEM((CHUNK, DH), jnp.float32),
        pltpu.VMEM_SHARED((N_NODES, DH), jnp.float32),
        pltpu.SemaphoreType.DMA,
        pltpu.SemaphoreType.DMA,
    ],
    compiler_params=_sc_params,
)
def _spmm_sc(row_hbm, col_hbm, xsl_hbm, xsh_hbm, xdl_hbm, xdh_hbm,
             ofl_hbm, obl_hbm, ofh_hbm, obh_hbm,
             row_v, col_v, buf_a, buf_b, acc, sem_a, sem_b):
    core = lax.axis_index("c")
    tile = lax.axis_index("s")

    pltpu.sync_copy(row_hbm.at[tile], row_v)
    pltpu.sync_copy(col_hbm.at[tile], col_v)

    def run_direction(src_hbm, g_idx, s_idx):
        def issue(c, buf, sem):
            pltpu.make_async_copy(src_hbm.at[g_idx.at[c]], buf, sem).start()

        def wait(c, buf, sem):
            pltpu.make_async_copy(src_hbm.at[g_idx.at[c]], buf, sem).wait()

        issue(0, buf_a, sem_a)

        @pl.loop(0, NCHUNK, step=2)
        def _(c):
            issue(c + 1, buf_b, sem_b)
            wait(c, buf_a, sem_a)
            pltpu.sync_copy(buf_a, acc.at[s_idx.at[c]], add=True)

            @pl.when(c + 2 < NCHUNK)
            def _():
                issue(c + 2, buf_a, sem_a)

            wait(c + 1, buf_b, sem_b)
            pltpu.sync_copy(buf_b, acc.at[s_idx.at[c + 1]], add=True)

    def _writeout(out_hbm):
        def fn(off, sz):
            pltpu.sync_copy(acc.at[pl.ds(off, sz)], buf_a.at[pl.ds(0, sz)])
            pltpu.sync_copy(buf_a.at[pl.ds(0, sz)], out_hbm.at[pl.ds(off, sz)])

        _for_tile_rows(tile, fn)

    def one_half(src_f, src_b, outf_hbm, outb_hbm):
        _fill_rows(buf_a, DH, 0.0)
        _for_tile_rows(
            tile,
            lambda off, sz: pltpu.sync_copy(buf_a.at[pl.ds(0, sz)],
                                            acc.at[pl.ds(off, sz)]))
        plsc.subcore_barrier()

        @pl.when(core == 0)
        def _():
            run_direction(src_f, col_v, row_v)

        @pl.when(core == 1)
        def _():
            run_direction(src_b, row_v, col_v)

        plsc.subcore_barrier()

        @pl.when(core == 0)
        def _():
            _writeout(outf_hbm)

        @pl.when(core == 1)
        def _():
            _writeout(outb_hbm)

    one_half(xsl_hbm, xdl_hbm, ofl_hbm, obl_hbm)
    one_half(xsh_hbm, xdh_hbm, ofh_hbm, obh_hbm)


def _matmul_scale_tc(x, w_sd, w_ds, degf, degb):
    def body(x_ref, wsd_ref, wds_ref, df_ref, db_ref, xsl_ref, xsh_ref,
             xdl_ref, xdh_ref):
        xb = x_ref[...]
        dn = (((1,), (1,)), ((), ()))
        xs = (ALPHA * _dinv(df_ref[:, 0:1])) * lax.dot_general(
            xb, wsd_ref[...], dn, preferred_element_type=jnp.float32)
        xd = ((1.0 - ALPHA) * _dinv(db_ref[:, 0:1])) * lax.dot_general(
            xb, wds_ref[...], dn, preferred_element_type=jnp.float32)
        xsl_ref[...] = xs[:, :DH]
        xsh_ref[...] = xs[:, DH:]
        xdl_ref[...] = xd[:, :DH]
        xdh_ref[...] = xd[:, DH:]

    blk = N_NODES // 10
    return pl.pallas_call(
        body,
        grid=(10,),
        in_specs=[
            pl.BlockSpec((blk, D), lambda i: (i, 0)),
            pl.BlockSpec((D, D), lambda i: (0, 0)),
            pl.BlockSpec((D, D), lambda i: (0, 0)),
            pl.BlockSpec((blk, HIST_W), lambda i: (i, 0)),
            pl.BlockSpec((blk, HIST_W), lambda i: (i, 0)),
        ],
        out_specs=[pl.BlockSpec((blk, DH), lambda i: (i, 0))] * 4,
        out_shape=[jax.ShapeDtypeStruct((N_NODES, DH), jnp.float32)] * 4,
    )(x, w_sd, w_ds, degf, degb)


def _dinv(deg_block):
    return jnp.where(deg_block > 0,
                     lax.rsqrt(jnp.maximum(deg_block, 1e-12)),
                     0.0)


def _combine_tc(afl, afh, abl, abh, degf, degb, bsd, bds):
    def body(afl_ref, afh_ref, abl_ref, abh_ref, df_ref, db_ref, bsd_ref,
             bds_ref, o_ref):
        bias = ALPHA * bsd_ref[0:1, :] + (1.0 - ALPHA) * bds_ref[0:1, :]
        dif = _dinv(df_ref[:, 0:1])
        dib = _dinv(db_ref[:, 0:1])
        af = jnp.concatenate([afl_ref[...], afh_ref[...]], axis=1)
        ab = jnp.concatenate([abl_ref[...], abh_ref[...]], axis=1)
        o_ref[...] = dif * af + dib * ab + bias

    blk = N_NODES // 10
    return pl.pallas_call(
        body,
        grid=(10,),
        in_specs=[
            pl.BlockSpec((blk, DH), lambda i: (i, 0)),
            pl.BlockSpec((blk, DH), lambda i: (i, 0)),
            pl.BlockSpec((blk, DH), lambda i: (i, 0)),
            pl.BlockSpec((blk, DH), lambda i: (i, 0)),
            pl.BlockSpec((blk, HIST_W), lambda i: (i, 0)),
            pl.BlockSpec((blk, HIST_W), lambda i: (i, 0)),
            pl.BlockSpec((8, D), lambda i: (0, 0)),
            pl.BlockSpec((8, D), lambda i: (0, 0)),
        ],
        out_specs=pl.BlockSpec((blk, D), lambda i: (i, 0)),
        out_shape=jax.ShapeDtypeStruct((N_NODES, D), jnp.float32),
    )(afl, afh, abl, abh, degf, degb, bsd, bds)


@jax.jit
def kernel(x, edge_index, W_sd, b_sd, W_ds, b_ds):
    row = edge_index[0].reshape(NS, NCHUNK, CHUNK)
    col = edge_index[1].reshape(NS, NCHUNK, CHUNK)

    degf, degb = _degree_sc(row, col)
    xsl, xsh, xdl, xdh = _matmul_scale_tc(x, W_sd, W_ds, degf, degb)
    afl, abl, afh, abh = _spmm_sc(row, col, xsl, xsh, xdl, xdh)

    bsd = jnp.broadcast_to(b_sd[None, :], (8, D))
    bds = jnp.broadcast_to(b_ds[None, :], (8, D))
    return _combine_tc(afl, afh, abl, abh, degf, degb, bsd, bds)


# speedup vs baseline: 1.4481x; 1.1643x over previous
import functools

import jax
import jax.numpy as jnp
from jax import lax
from jax.experimental import pallas as pl
from jax.experimental.pallas import tpu as pltpu
from jax.experimental.pallas import tpu_sc as plsc

N_NODES = 10000
N_EDGES = 320000
D = 128
ALPHA = 0.5

NS = 16
CHUNK = 128
NCHUNK = 160
EDGES_PAD = NS * NCHUNK * CHUNK
PAD_ROWS = 240
N_PAD = N_NODES + PAD_ROWS
INIT_ROWS_PER_TILE = N_PAD // NS
ROWS_PER_TILE = 624
ROWS_TAIL = N_NODES - NS * ROWS_PER_TILE
HIST_W = 16

_mesh = plsc.VectorSubcoreMesh(core_axis_name="c", subcore_axis_name="s")
_sc_params = pltpu.CompilerParams(use_tc_tiling_on_sc=False)


def _init_tile_rows(tile, fn):
    base = tile * INIT_ROWS_PER_TILE

    @pl.loop(0, INIT_ROWS_PER_TILE // CHUNK)
    def _(c):
        fn(base + c * CHUNK, CHUNK)


def _out_tile_rows(tile, fn):
    base = tile * ROWS_PER_TILE
    n_full = ROWS_PER_TILE // CHUNK
    rem = ROWS_PER_TILE % CHUNK

    @pl.loop(0, n_full)
    def _(c):
        fn(base + c * CHUNK, CHUNK)

    if rem:
        fn(base + n_full * CHUNK, rem)

    @pl.when(tile == NS - 1)
    def _():
        fn(NS * ROWS_PER_TILE, ROWS_TAIL)


def _fill_rows(buf, width, value):
    @pl.loop(0, CHUNK)
    def _(i):
        for j in range(width // 16):
            buf[i, pl.ds(j * 16, 16)] = jnp.full((16,), value, jnp.float32)


@functools.partial(
    pl.kernel,
    out_type=(
        jax.ShapeDtypeStruct((N_NODES, HIST_W), jnp.float32),
        jax.ShapeDtypeStruct((N_NODES, HIST_W), jnp.float32),
    ),
    mesh=_mesh,
    scratch_types=[
        pltpu.VMEM((NCHUNK, CHUNK), jnp.int32),
        pltpu.VMEM((CHUNK, HIST_W), jnp.float32),
        pltpu.VMEM((CHUNK, HIST_W), jnp.float32),
        pltpu.VMEM_SHARED((N_PAD, HIST_W), jnp.float32),
    ],
    compiler_params=_sc_params,
)
def _degree_sc(row_hbm, col_hbm, degf_hbm, degb_hbm, idx_v, ones_v, zero_v,
               hist):
    core = lax.axis_index("c")
    tile = lax.axis_index("s")

    _fill_rows(ones_v, HIST_W, 1.0)
    _fill_rows(zero_v, HIST_W, 0.0)
    _init_tile_rows(
        tile,
        lambda off, sz: pltpu.sync_copy(zero_v.at[pl.ds(0, sz)],
                                        hist.at[pl.ds(off, sz)]))

    @pl.when(core == 0)
    def _():
        pltpu.sync_copy(row_hbm.at[tile], idx_v)

    @pl.when(core == 1)
    def _():
        pltpu.sync_copy(col_hbm.at[tile], idx_v)

    plsc.subcore_barrier()

    @pl.loop(0, NCHUNK)
    def _(c):
        pltpu.sync_copy(ones_v, hist.at[idx_v.at[c]], add=True)

    plsc.subcore_barrier()

    def _writeout(out_hbm):
        def fn(off, sz):
            pltpu.sync_copy(hist.at[pl.ds(off, sz)], zero_v.at[pl.ds(0, sz)])
            pltpu.sync_copy(zero_v.at[pl.ds(0, sz)], out_hbm.at[pl.ds(off, sz)])

        _out_tile_rows(tile, fn)

    @pl.when(core == 0)
    def _():
        _writeout(degf_hbm)

    @pl.when(core == 1)
    def _():
        _writeout(degb_hbm)


DH = D // 2


@functools.partial(
    pl.kernel,
    out_type=(
        jax.ShapeDtypeStruct((N_NODES, DH), jnp.float32),
        jax.ShapeDtypeStruct((N_NODES, DH), jnp.float32),
        jax.ShapeDtypeStruct((N_NODES, DH), jnp.float32),
        jax.ShapeDtypeStruct((N_NODES, DH), jnp.float32),
    ),
    mesh=_mesh,
    scratch_types=[
        pltpu.VMEM((NCHUNK, CHUNK), jnp.int32),
        pltpu.VMEM((NCHUNK, CHUNK), jnp.int32),
        pltpu.VMEM((CHUNK, DH), jnp.float32),
        pltpu.VMEM((CHUNK, DH), jnp.float32),
        pltpu.VMEM_SHARED((N_PAD, DH), jnp.float32),
        pltpu.SemaphoreType.DMA,
        pltpu.SemaphoreType.DMA,
    ],
    compiler_params=_sc_params,
)
def _spmm_sc(row_hbm, col_hbm, xsl_hbm, xsh_hbm, xdl_hbm, xdh_hbm,
             ofl_hbm, obl_hbm, ofh_hbm, obh_hbm,
             row_v, col_v, buf_a, buf_b, acc, sem_a, sem_b):
    core = lax.axis_index("c")
    tile = lax.axis_index("s")

    pltpu.sync_copy(row_hbm.at[tile], row_v)
    pltpu.sync_copy(col_hbm.at[tile], col_v)

    def run_direction(src_hbm, g_idx, s_idx):
        def issue(c, buf, sem):
            pltpu.make_async_copy(src_hbm.at[g_idx.at[c]], buf, sem).start()

        def wait(c, buf, sem):
            pltpu.make_async_copy(src_hbm.at[g_idx.at[c]], buf, sem).wait()

        issue(0, buf_a, sem_a)

        @pl.loop(0, NCHUNK, step=2)
        def _(c):
            issue(c + 1, buf_b, sem_b)
            wait(c, buf_a, sem_a)
            pltpu.sync_copy(buf_a, acc.at[s_idx.at[c]], add=True)

            @pl.when(c + 2 < NCHUNK)
            def _():
                issue(c + 2, buf_a, sem_a)

            wait(c + 1, buf_b, sem_b)
            pltpu.sync_copy(buf_b, acc.at[s_idx.at[c + 1]], add=True)

    def _writeout(out_hbm):
        def fn(off, sz):
            pltpu.sync_copy(acc.at[pl.ds(off, sz)], buf_a.at[pl.ds(0, sz)])
            pltpu.sync_copy(buf_a.at[pl.ds(0, sz)], out_hbm.at[pl.ds(off, sz)])

        _out_tile_rows(tile, fn)

    def one_half(src_f, src_b, outf_hbm, outb_hbm):
        _fill_rows(buf_a, DH, 0.0)
        _init_tile_rows(
            tile,
            lambda off, sz: pltpu.sync_copy(buf_a.at[pl.ds(0, sz)],
                                            acc.at[pl.ds(off, sz)]))
        plsc.subcore_barrier()

        @pl.when(core == 0)
        def _():
            run_direction(src_f, col_v, row_v)

        @pl.when(core == 1)
        def _():
            run_direction(src_b, row_v, col_v)

        plsc.subcore_barrier()

        @pl.when(core == 0)
        def _():
            _writeout(outf_hbm)

        @pl.when(core == 1)
        def _():
            _writeout(outb_hbm)

    one_half(xsl_hbm, xdl_hbm, ofl_hbm, obl_hbm)
    one_half(xsh_hbm, xdh_hbm, ofh_hbm, obh_hbm)


def _matmul_scale_tc(x, w_sd, w_ds, degf, degb):
    def body(x_ref, wsd_ref, wds_ref, df_ref, db_ref, xsl_ref, xsh_ref,
             xdl_ref, xdh_ref):
        xb = x_ref[...]
        dn = (((1,), (1,)), ((), ()))
        xs = (ALPHA * _dinv(df_ref[:, 0:1])) * lax.dot_general(
            xb, wsd_ref[...], dn, preferred_element_type=jnp.float32)
        xd = ((1.0 - ALPHA) * _dinv(db_ref[:, 0:1])) * lax.dot_general(
            xb, wds_ref[...], dn, preferred_element_type=jnp.float32)
        xsl_ref[...] = xs[:, :DH]
        xsh_ref[...] = xs[:, DH:]
        xdl_ref[...] = xd[:, :DH]
        xdh_ref[...] = xd[:, DH:]

    blk = N_NODES // 10
    return pl.pallas_call(
        body,
        grid=(10,),
        in_specs=[
            pl.BlockSpec((blk, D), lambda i: (i, 0)),
            pl.BlockSpec((D, D), lambda i: (0, 0)),
            pl.BlockSpec((D, D), lambda i: (0, 0)),
            pl.BlockSpec((blk, HIST_W), lambda i: (i, 0)),
            pl.BlockSpec((blk, HIST_W), lambda i: (i, 0)),
        ],
        out_specs=[pl.BlockSpec((blk, DH), lambda i: (i, 0))] * 4,
        out_shape=[jax.ShapeDtypeStruct((N_PAD, DH), jnp.float32)] * 4,
    )(x, w_sd, w_ds, degf, degb)


def _dinv(deg_block):
    return jnp.where(deg_block > 0,
                     lax.rsqrt(jnp.maximum(deg_block, 1e-12)),
                     0.0)


def _combine_tc(afl, afh, abl, abh, degf, degb, bsd, bds):
    def body(afl_ref, afh_ref, abl_ref, abh_ref, df_ref, db_ref, bsd_ref,
             bds_ref, o_ref):
        bias = ALPHA * bsd_ref[0:1, :] + (1.0 - ALPHA) * bds_ref[0:1, :]
        dif = _dinv(df_ref[:, 0:1])
        dib = _dinv(db_ref[:, 0:1])
        af = jnp.concatenate([afl_ref[...], afh_ref[...]], axis=1)
        ab = jnp.concatenate([abl_ref[...], abh_ref[...]], axis=1)
        o_ref[...] = dif * af + dib * ab + bias

    blk = N_NODES // 10
    return pl.pallas_call(
        body,
        grid=(10,),
        in_specs=[
            pl.BlockSpec((blk, DH), lambda i: (i, 0)),
            pl.BlockSpec((blk, DH), lambda i: (i, 0)),
            pl.BlockSpec((blk, DH), lambda i: (i, 0)),
            pl.BlockSpec((blk, DH), lambda i: (i, 0)),
            pl.BlockSpec((blk, HIST_W), lambda i: (i, 0)),
            pl.BlockSpec((blk, HIST_W), lambda i: (i, 0)),
            pl.BlockSpec((8, D), lambda i: (0, 0)),
            pl.BlockSpec((8, D), lambda i: (0, 0)),
        ],
        out_specs=pl.BlockSpec((blk, D), lambda i: (i, 0)),
        out_shape=jax.ShapeDtypeStruct((N_NODES, D), jnp.float32),
    )(afl, afh, abl, abh, degf, degb, bsd, bds)


@jax.jit
def kernel(x, edge_index, W_sd, b_sd, W_ds, b_ds):
    pad = N_NODES + (jnp.arange(EDGES_PAD - N_EDGES, dtype=jnp.int32)
                     % PAD_ROWS)
    row = jnp.concatenate([edge_index[0], pad]).reshape(NS, NCHUNK, CHUNK)
    col = jnp.concatenate([edge_index[1], pad]).reshape(NS, NCHUNK, CHUNK)

    degf, degb = _degree_sc(row, col)
    xsl, xsh, xdl, xdh = _matmul_scale_tc(x, W_sd, W_ds, degf, degb)
    afl, abl, afh, abh = _spmm_sc(row, col, xsl, xsh, xdl, xdh)

    bsd = jnp.broadcast_to(b_sd[None, :], (8, D))
    bds = jnp.broadcast_to(b_ds[None, :], (8, D))
    return _combine_tc(afl, afh, abl, abh, degf, degb, bsd, bds)


# speedup vs baseline: 1.4789x; 1.0213x over previous
import functools

import jax
import jax.numpy as jnp
from jax import lax
from jax.experimental import pallas as pl
from jax.experimental.pallas import tpu as pltpu
from jax.experimental.pallas import tpu_sc as plsc

N_NODES = 10000
N_EDGES = 320000
D = 128
ALPHA = 0.5

NS = 16
CHUNK = 128
NCHUNK = 160
EDGES_PAD = NS * NCHUNK * CHUNK
PAD_ROWS = 240
N_PAD = N_NODES + PAD_ROWS
INIT_ROWS_PER_TILE = N_PAD // NS
ROWS_PER_TILE = 624
ROWS_TAIL = N_NODES - NS * ROWS_PER_TILE
HIST_W = 16

_mesh = plsc.VectorSubcoreMesh(core_axis_name="c", subcore_axis_name="s")
_sc_params = pltpu.CompilerParams(use_tc_tiling_on_sc=False)


def _init_tile_rows(tile, fn):
    base = tile * INIT_ROWS_PER_TILE

    @pl.loop(0, INIT_ROWS_PER_TILE // CHUNK)
    def _(c):
        fn(base + c * CHUNK, CHUNK)


def _out_tile_rows(tile, fn):
    base = tile * ROWS_PER_TILE
    n_full = ROWS_PER_TILE // CHUNK
    rem = ROWS_PER_TILE % CHUNK

    @pl.loop(0, n_full)
    def _(c):
        fn(base + c * CHUNK, CHUNK)

    if rem:
        fn(base + n_full * CHUNK, rem)

    @pl.when(tile == NS - 1)
    def _():
        fn(NS * ROWS_PER_TILE, ROWS_TAIL)


def _fill_rows(buf, width, value):
    @pl.loop(0, CHUNK)
    def _(i):
        for j in range(width // 16):
            buf[i, pl.ds(j * 16, 16)] = jnp.full((16,), value, jnp.float32)


@functools.partial(
    pl.kernel,
    out_type=(
        jax.ShapeDtypeStruct((N_NODES, HIST_W), jnp.float32),
        jax.ShapeDtypeStruct((N_NODES, HIST_W), jnp.float32),
    ),
    mesh=_mesh,
    scratch_types=[
        pltpu.VMEM((NCHUNK, CHUNK), jnp.int32),
        pltpu.VMEM((CHUNK, HIST_W), jnp.float32),
        pltpu.VMEM((CHUNK, HIST_W), jnp.float32),
        pltpu.VMEM_SHARED((N_PAD, HIST_W), jnp.float32),
        pltpu.SemaphoreType.DMA,
    ],
    compiler_params=_sc_params,
)
def _degree_sc(row_hbm, col_hbm, degf_hbm, degb_hbm, idx_v, ones_v, zero_v,
               hist, sem):
    core = lax.axis_index("c")
    tile = lax.axis_index("s")

    _fill_rows(ones_v, HIST_W, 1.0)
    _fill_rows(zero_v, HIST_W, 0.0)
    _init_tile_rows(
        tile,
        lambda off, sz: pltpu.sync_copy(zero_v.at[pl.ds(0, sz)],
                                        hist.at[pl.ds(off, sz)]))

    @pl.when(core == 0)
    def _():
        pltpu.sync_copy(row_hbm.at[tile], idx_v)

    @pl.when(core == 1)
    def _():
        pltpu.sync_copy(col_hbm.at[tile], idx_v)

    plsc.subcore_barrier()

    DEPTH = 4
    for k in range(DEPTH):
        pltpu.async_copy(ones_v, hist.at[idx_v.at[k]], sem, add=True)

    @pl.loop(0, NCHUNK)
    def _(c):
        @pl.when(c + DEPTH < NCHUNK)
        def _():
            pltpu.async_copy(ones_v, hist.at[idx_v.at[c + DEPTH]], sem,
                             add=True)

        pltpu.make_async_copy(ones_v, hist.at[idx_v.at[c]], sem).wait()

    plsc.subcore_barrier()

    def _writeout(out_hbm):
        def fn(off, sz):
            pltpu.sync_copy(hist.at[pl.ds(off, sz)], zero_v.at[pl.ds(0, sz)])
            pltpu.sync_copy(zero_v.at[pl.ds(0, sz)], out_hbm.at[pl.ds(off, sz)])

        _out_tile_rows(tile, fn)

    @pl.when(core == 0)
    def _():
        _writeout(degf_hbm)

    @pl.when(core == 1)
    def _():
        _writeout(degb_hbm)


DH = D // 2


@functools.partial(
    pl.kernel,
    out_type=(
        jax.ShapeDtypeStruct((N_NODES, DH), jnp.float32),
        jax.ShapeDtypeStruct((N_NODES, DH), jnp.float32),
        jax.ShapeDtypeStruct((N_NODES, DH), jnp.float32),
        jax.ShapeDtypeStruct((N_NODES, DH), jnp.float32),
    ),
    mesh=_mesh,
    scratch_types=[
        pltpu.VMEM((NCHUNK, CHUNK), jnp.int32),
        pltpu.VMEM((NCHUNK, CHUNK), jnp.int32),
        pltpu.VMEM((CHUNK, DH), jnp.float32),
        pltpu.VMEM((CHUNK, DH), jnp.float32),
        pltpu.VMEM_SHARED((N_PAD, DH), jnp.float32),
        pltpu.SemaphoreType.DMA,
        pltpu.SemaphoreType.DMA,
    ],
    compiler_params=_sc_params,
)
def _spmm_sc(row_hbm, col_hbm, xsl_hbm, xsh_hbm, xdl_hbm, xdh_hbm,
             ofl_hbm, obl_hbm, ofh_hbm, obh_hbm,
             row_v, col_v, buf_a, buf_b, acc, sem_a, sem_b):
    core = lax.axis_index("c")
    tile = lax.axis_index("s")

    pltpu.sync_copy(row_hbm.at[tile], row_v)
    pltpu.sync_copy(col_hbm.at[tile], col_v)

    def run_direction(src_hbm, g_idx, s_idx):
        def issue(c, buf, sem):
            pltpu.make_async_copy(src_hbm.at[g_idx.at[c]], buf, sem).start()

        def wait(c, buf, sem):
            pltpu.make_async_copy(src_hbm.at[g_idx.at[c]], buf, sem).wait()

        issue(0, buf_a, sem_a)

        @pl.loop(0, NCHUNK, step=2)
        def _(c):
            issue(c + 1, buf_b, sem_b)
            wait(c, buf_a, sem_a)
            pltpu.sync_copy(buf_a, acc.at[s_idx.at[c]], add=True)

            @pl.when(c + 2 < NCHUNK)
            def _():
                issue(c + 2, buf_a, sem_a)

            wait(c + 1, buf_b, sem_b)
            pltpu.sync_copy(buf_b, acc.at[s_idx.at[c + 1]], add=True)

    def _writeout(out_hbm):
        def fn(off, sz):
            pltpu.sync_copy(acc.at[pl.ds(off, sz)], buf_a.at[pl.ds(0, sz)])
            pltpu.sync_copy(buf_a.at[pl.ds(0, sz)], out_hbm.at[pl.ds(off, sz)])

        _out_tile_rows(tile, fn)

    def one_half(src_f, src_b, outf_hbm, outb_hbm):
        _fill_rows(buf_a, DH, 0.0)
        _init_tile_rows(
            tile,
            lambda off, sz: pltpu.sync_copy(buf_a.at[pl.ds(0, sz)],
                                            acc.at[pl.ds(off, sz)]))
        plsc.subcore_barrier()

        @pl.when(core == 0)
        def _():
            run_direction(src_f, col_v, row_v)

        @pl.when(core == 1)
        def _():
            run_direction(src_b, row_v, col_v)

        plsc.subcore_barrier()

        @pl.when(core == 0)
        def _():
            _writeout(outf_hbm)

        @pl.when(core == 1)
        def _():
            _writeout(outb_hbm)

    one_half(xsl_hbm, xdl_hbm, ofl_hbm, obl_hbm)
    one_half(xsh_hbm, xdh_hbm, ofh_hbm, obh_hbm)


def _matmul_scale_tc(x, w_sd, w_ds, degf, degb):
    def body(x_ref, wsd_ref, wds_ref, df_ref, db_ref, xsl_ref, xsh_ref,
             xdl_ref, xdh_ref):
        xb = x_ref[...]
        dn = (((1,), (1,)), ((), ()))
        xs = (ALPHA * _dinv(df_ref[:, 0:1])) * lax.dot_general(
            xb, wsd_ref[...], dn, preferred_element_type=jnp.float32)
        xd = ((1.0 - ALPHA) * _dinv(db_ref[:, 0:1])) * lax.dot_general(
            xb, wds_ref[...], dn, preferred_element_type=jnp.float32)
        xsl_ref[...] = xs[:, :DH]
        xsh_ref[...] = xs[:, DH:]
        xdl_ref[...] = xd[:, :DH]
        xdh_ref[...] = xd[:, DH:]

    blk = N_NODES // 10
    return pl.pallas_call(
        body,
        grid=(10,),
        in_specs=[
            pl.BlockSpec((blk, D), lambda i: (i, 0)),
            pl.BlockSpec((D, D), lambda i: (0, 0)),
            pl.BlockSpec((D, D), lambda i: (0, 0)),
            pl.BlockSpec((blk, HIST_W), lambda i: (i, 0)),
            pl.BlockSpec((blk, HIST_W), lambda i: (i, 0)),
        ],
        out_specs=[pl.BlockSpec((blk, DH), lambda i: (i, 0))] * 4,
        out_shape=[jax.ShapeDtypeStruct((N_PAD, DH), jnp.float32)] * 4,
    )(x, w_sd, w_ds, degf, degb)


def _dinv(deg_block):
    return jnp.where(deg_block > 0,
                     lax.rsqrt(jnp.maximum(deg_block, 1e-12)),
                     0.0)


def _combine_tc(afl, afh, abl, abh, degf, degb, bsd, bds):
    def body(afl_ref, afh_ref, abl_ref, abh_ref, df_ref, db_ref, bsd_ref,
             bds_ref, o_ref):
        bias = ALPHA * bsd_ref[0:1, :] + (1.0 - ALPHA) * bds_ref[0:1, :]
        dif = _dinv(df_ref[:, 0:1])
        dib = _dinv(db_ref[:, 0:1])
        af = jnp.concatenate([afl_ref[...], afh_ref[...]], axis=1)
        ab = jnp.concatenate([abl_ref[...], abh_ref[...]], axis=1)
        o_ref[...] = dif * af + dib * ab + bias

    blk = N_NODES // 10
    return pl.pallas_call(
        body,
        grid=(10,),
        in_specs=[
            pl.BlockSpec((blk, DH), lambda i: (i, 0)),
            pl.BlockSpec((blk, DH), lambda i: (i, 0)),
            pl.BlockSpec((blk, DH), lambda i: (i, 0)),
            pl.BlockSpec((blk, DH), lambda i: (i, 0)),
            pl.BlockSpec((blk, HIST_W), lambda i: (i, 0)),
            pl.BlockSpec((blk, HIST_W), lambda i: (i, 0)),
            pl.BlockSpec((8, D), lambda i: (0, 0)),
            pl.BlockSpec((8, D), lambda i: (0, 0)),
        ],
        out_specs=pl.BlockSpec((blk, D), lambda i: (i, 0)),
        out_shape=jax.ShapeDtypeStruct((N_NODES, D), jnp.float32),
    )(afl, afh, abl, abh, degf, degb, bsd, bds)


@jax.jit
def kernel(x, edge_index, W_sd, b_sd, W_ds, b_ds):
    pad = N_NODES + (jnp.arange(EDGES_PAD - N_EDGES, dtype=jnp.int32)
                     % PAD_ROWS)
    row = jnp.concatenate([edge_index[0], pad]).reshape(NS, NCHUNK, CHUNK)
    col = jnp.concatenate([edge_index[1], pad]).reshape(NS, NCHUNK, CHUNK)

    degf, degb = _degree_sc(row, col)
    xsl, xsh, xdl, xdh = _matmul_scale_tc(x, W_sd, W_ds, degf, degb)
    afl, abl, afh, abh = _spmm_sc(row, col, xsl, xsh, xdl, xdh)

    bsd = jnp.broadcast_to(b_sd[None, :], (8, D))
    bds = jnp.broadcast_to(b_ds[None, :], (8, D))
    return _combine_tc(afl, afh, abl, abh, degf, degb, bsd, bds)


# speedup vs baseline: 1.5544x; 1.0511x over previous
import functools

import jax
import jax.numpy as jnp
from jax import lax
from jax.experimental import pallas as pl
from jax.experimental.pallas import tpu as pltpu
from jax.experimental.pallas import tpu_sc as plsc

N_NODES = 10000
N_EDGES = 320000
D = 128
ALPHA = 0.5

NS = 16
CHUNK = 128
NCHUNK = 160
EDGES_PAD = NS * NCHUNK * CHUNK
PAD_ROWS = 240
N_PAD = N_NODES + PAD_ROWS
INIT_ROWS_PER_TILE = N_PAD // NS
ROWS_PER_TILE = 624
ROWS_TAIL = N_NODES - NS * ROWS_PER_TILE
HIST_W = 16

_mesh = plsc.VectorSubcoreMesh(core_axis_name="c", subcore_axis_name="s")
_sc_params = pltpu.CompilerParams(use_tc_tiling_on_sc=False)


def _init_tile_rows(tile, fn):
    base = tile * INIT_ROWS_PER_TILE

    @pl.loop(0, INIT_ROWS_PER_TILE // CHUNK)
    def _(c):
        fn(base + c * CHUNK, CHUNK)


def _out_tile_rows(tile, fn):
    base = tile * ROWS_PER_TILE
    n_full = ROWS_PER_TILE // CHUNK
    rem = ROWS_PER_TILE % CHUNK

    @pl.loop(0, n_full)
    def _(c):
        fn(base + c * CHUNK, CHUNK)

    if rem:
        fn(base + n_full * CHUNK, rem)

    @pl.when(tile == NS - 1)
    def _():
        fn(NS * ROWS_PER_TILE, ROWS_TAIL)


def _fill_rows(buf, width, value):
    @pl.loop(0, CHUNK)
    def _(i):
        for j in range(width // 16):
            buf[i, pl.ds(j * 16, 16)] = jnp.full((16,), value, jnp.float32)


@functools.partial(
    pl.kernel,
    out_type=(
        jax.ShapeDtypeStruct((N_PAD, HIST_W), jnp.float32),
        jax.ShapeDtypeStruct((N_PAD, HIST_W), jnp.float32),
    ),
    mesh=_mesh,
    scratch_types=[
        pltpu.VMEM((NCHUNK, CHUNK), jnp.int32),
        pltpu.VMEM((CHUNK, HIST_W), jnp.float32),
        pltpu.VMEM((CHUNK, HIST_W), jnp.float32),
        pltpu.VMEM_SHARED((N_PAD, HIST_W), jnp.float32),
        pltpu.SemaphoreType.DMA,
    ],
    compiler_params=_sc_params,
)
def _degree_sc(row_hbm, col_hbm, degf_hbm, degb_hbm, idx_v, ones_v, zero_v,
               hist, sem):
    core = lax.axis_index("c")
    tile = lax.axis_index("s")

    _fill_rows(ones_v, HIST_W, 1.0)
    _fill_rows(zero_v, HIST_W, 0.0)
    _init_tile_rows(
        tile,
        lambda off, sz: pltpu.sync_copy(zero_v.at[pl.ds(0, sz)],
                                        hist.at[pl.ds(off, sz)]))

    @pl.when(core == 0)
    def _():
        pltpu.sync_copy(row_hbm.at[tile], idx_v)

    @pl.when(core == 1)
    def _():
        pltpu.sync_copy(col_hbm.at[tile], idx_v)

    plsc.subcore_barrier()

    DEPTH = 4
    for k in range(DEPTH):
        pltpu.async_copy(ones_v, hist.at[idx_v.at[k]], sem, add=True)

    @pl.loop(0, NCHUNK)
    def _(c):
        @pl.when(c + DEPTH < NCHUNK)
        def _():
            pltpu.async_copy(ones_v, hist.at[idx_v.at[c + DEPTH]], sem,
                             add=True)

        pltpu.make_async_copy(ones_v, hist.at[idx_v.at[c]], sem).wait()

    plsc.subcore_barrier()

    def _writeout(out_hbm):
        def fn(off, sz):
            pltpu.sync_copy(hist.at[pl.ds(off, sz)], zero_v.at[pl.ds(0, sz)])
            pltpu.sync_copy(zero_v.at[pl.ds(0, sz)], out_hbm.at[pl.ds(off, sz)])

        _init_tile_rows(tile, fn)

    @pl.when(core == 0)
    def _():
        _writeout(degf_hbm)

    @pl.when(core == 1)
    def _():
        _writeout(degb_hbm)


DH = D // 2


@functools.partial(
    pl.kernel,
    out_type=(
        jax.ShapeDtypeStruct((N_NODES, DH), jnp.float32),
        jax.ShapeDtypeStruct((N_NODES, DH), jnp.float32),
        jax.ShapeDtypeStruct((N_NODES, DH), jnp.float32),
        jax.ShapeDtypeStruct((N_NODES, DH), jnp.float32),
    ),
    mesh=_mesh,
    scratch_types=[
        pltpu.VMEM((NCHUNK, CHUNK), jnp.int32),
        pltpu.VMEM((NCHUNK, CHUNK), jnp.int32),
        pltpu.VMEM((CHUNK, DH), jnp.float32),
        pltpu.VMEM((CHUNK, DH), jnp.float32),
        pltpu.VMEM_SHARED((N_PAD, DH), jnp.float32),
        pltpu.SemaphoreType.DMA,
        pltpu.SemaphoreType.DMA,
    ],
    compiler_params=_sc_params,
)
def _spmm_sc(row_hbm, col_hbm, xsl_hbm, xsh_hbm, xdl_hbm, xdh_hbm,
             ofl_hbm, obl_hbm, ofh_hbm, obh_hbm,
             row_v, col_v, buf_a, buf_b, acc, sem_a, sem_b):
    core = lax.axis_index("c")
    tile = lax.axis_index("s")

    pltpu.sync_copy(row_hbm.at[tile], row_v)
    pltpu.sync_copy(col_hbm.at[tile], col_v)

    def run_direction(src_hbm, g_idx, s_idx):
        def issue(c, buf, sem):
            pltpu.make_async_copy(src_hbm.at[g_idx.at[c]], buf, sem).start()

        def wait(c, buf, sem):
            pltpu.make_async_copy(src_hbm.at[g_idx.at[c]], buf, sem).wait()

        issue(0, buf_a, sem_a)

        @pl.loop(0, NCHUNK, step=2)
        def _(c):
            issue(c + 1, buf_b, sem_b)
            wait(c, buf_a, sem_a)
            pltpu.sync_copy(buf_a, acc.at[s_idx.at[c]], add=True)

            @pl.when(c + 2 < NCHUNK)
            def _():
                issue(c + 2, buf_a, sem_a)

            wait(c + 1, buf_b, sem_b)
            pltpu.sync_copy(buf_b, acc.at[s_idx.at[c + 1]], add=True)

    def _writeout(out_hbm):
        def fn(off, sz):
            pltpu.sync_copy(acc.at[pl.ds(off, sz)], buf_a.at[pl.ds(0, sz)])
            pltpu.sync_copy(buf_a.at[pl.ds(0, sz)], out_hbm.at[pl.ds(off, sz)])

        _out_tile_rows(tile, fn)

    def one_half(src_f, src_b, outf_hbm, outb_hbm):
        _fill_rows(buf_a, DH, 0.0)
        _init_tile_rows(
            tile,
            lambda off, sz: pltpu.sync_copy(buf_a.at[pl.ds(0, sz)],
                                            acc.at[pl.ds(off, sz)]))
        plsc.subcore_barrier()

        @pl.when(core == 0)
        def _():
            run_direction(src_f, col_v, row_v)

        @pl.when(core == 1)
        def _():
            run_direction(src_b, row_v, col_v)

        plsc.subcore_barrier()

        @pl.when(core == 0)
        def _():
            _writeout(outf_hbm)

        @pl.when(core == 1)
        def _():
            _writeout(outb_hbm)

    one_half(xsl_hbm, xdl_hbm, ofl_hbm, obl_hbm)
    one_half(xsh_hbm, xdh_hbm, ofh_hbm, obh_hbm)


def _matmul_scale_tc(x, w_sd, w_ds, degf, degb):
    blk = N_NODES // 5
    pblk = blk // 2

    def body(x_ref, wsd_ref, wds_ref, df_ref, db_ref, xsl_ref, xsh_ref,
             xdl_ref, xdh_ref):
        xb = x_ref[...]
        dn = (((1,), (1,)), ((), ()))
        dif = _dinv(df_ref[:, 0:1])
        dib = _dinv(db_ref[:, 0:1])
        xs = (ALPHA * dif) * lax.dot_general(
            xb, wsd_ref[...], dn, preferred_element_type=jnp.float32)
        xd = ((1.0 - ALPHA) * dib) * lax.dot_general(
            xb, wds_ref[...], dn, preferred_element_type=jnp.float32)
        x3s = xs.reshape(pblk, 2, D)
        x3d = xd.reshape(pblk, 2, D)
        xsl_ref[...] = jnp.concatenate([x3s[:, 0, :DH], x3s[:, 1, :DH]], 1)
        xsh_ref[...] = jnp.concatenate([x3s[:, 0, DH:], x3s[:, 1, DH:]], 1)
        xdl_ref[...] = jnp.concatenate([x3d[:, 0, :DH], x3d[:, 1, :DH]], 1)
        xdh_ref[...] = jnp.concatenate([x3d[:, 0, DH:], x3d[:, 1, DH:]], 1)

    return pl.pallas_call(
        body,
        grid=(5,),
        in_specs=[
            pl.BlockSpec((blk, D), lambda i: (i, 0)),
            pl.BlockSpec((D, D), lambda i: (0, 0)),
            pl.BlockSpec((D, D), lambda i: (0, 0)),
            pl.BlockSpec((blk, HIST_W), lambda i: (i, 0)),
            pl.BlockSpec((blk, HIST_W), lambda i: (i, 0)),
        ],
        out_specs=[pl.BlockSpec((pblk, D), lambda i: (i, 0))] * 4,
        out_shape=[jax.ShapeDtypeStruct((N_PAD // 2, D), jnp.float32)] * 4,
    )(x, w_sd, w_ds, degf, degb)


def _dinv(deg_block):
    return jnp.where(deg_block > 0,
                     lax.rsqrt(jnp.maximum(deg_block, 1e-12)),
                     0.0)


def _combine_tc(afl_pk, afh_pk, abl_pk, abh_pk, degf, degb, bsd, bds):
    blk = N_NODES // 5
    pblk = blk // 2

    def body(afl_ref, afh_ref, abl_ref, abh_ref, df_ref, db_ref, bsd_ref,
             bds_ref, o_ref):
        bias = ALPHA * bsd_ref[0:1, :] + (1.0 - ALPHA) * bds_ref[0:1, :]
        dif = _dinv(df_ref[:, 0:1])
        dib = _dinv(db_ref[:, 0:1])
        def unpack(pk):
            return jnp.stack([pk[:, :DH], pk[:, DH:]], axis=1).reshape(blk,
                                                                       DH)

        af = jnp.concatenate([unpack(afl_ref[...]), unpack(afh_ref[...])],
                             axis=1)
        ab = jnp.concatenate([unpack(abl_ref[...]), unpack(abh_ref[...])],
                             axis=1)
        o_ref[...] = dif * af + dib * ab + bias

    return pl.pallas_call(
        body,
        grid=(5,),
        in_specs=[
            pl.BlockSpec((pblk, D), lambda i: (i, 0)),
            pl.BlockSpec((pblk, D), lambda i: (i, 0)),
            pl.BlockSpec((pblk, D), lambda i: (i, 0)),
            pl.BlockSpec((pblk, D), lambda i: (i, 0)),
            pl.BlockSpec((blk, HIST_W), lambda i: (i, 0)),
            pl.BlockSpec((blk, HIST_W), lambda i: (i, 0)),
            pl.BlockSpec((8, D), lambda i: (0, 0)),
            pl.BlockSpec((8, D), lambda i: (0, 0)),
        ],
        out_specs=pl.BlockSpec((blk, D), lambda i: (i, 0)),
        out_shape=jax.ShapeDtypeStruct((N_NODES, D), jnp.float32),
    )(afl_pk, afh_pk, abl_pk, abh_pk, degf, degb, bsd, bds)


@jax.jit
def kernel(x, edge_index, W_sd, b_sd, W_ds, b_ds):
    pad = N_NODES + (jnp.arange(EDGES_PAD - N_EDGES, dtype=jnp.int32)
                     % PAD_ROWS)
    row = jnp.concatenate([edge_index[0], pad]).reshape(NS, NCHUNK, CHUNK)
    col = jnp.concatenate([edge_index[1], pad]).reshape(NS, NCHUNK, CHUNK)

    degf, degb = _degree_sc(row, col)
    xsl, xsh, xdl, xdh = (t.reshape(N_PAD, DH) for t in _matmul_scale_tc(
        x, W_sd, W_ds, degf, degb))
    afl, abl, afh, abh = _spmm_sc(row, col, xsl, xsh, xdl, xdh)

    bsd = jnp.broadcast_to(b_sd[None, :], (8, D))
    bds = jnp.broadcast_to(b_ds[None, :], (8, D))
    return _combine_tc(afl.reshape(N_NODES // 2, D),
                       afh.reshape(N_NODES // 2, D),
                       abl.reshape(N_NODES // 2, D),
                       abh.reshape(N_NODES // 2, D),
                       degf, degb, bsd, bds)
